# baseline (speedup 1.0000x reference)
import functools

import jax
import jax.numpy as jnp
from jax import lax
from jax.experimental import pallas as pl
from jax.experimental.pallas import tpu as pltpu
from jax.experimental.pallas import tpu_sc as plsc

M = 1000000
N_REL = 1000
D = 64
B = 16384
ALPHA = 0.2
GAMMA = 12.0

NC = 2
NS = 16
NW = NC * NS
BPW = B // NW
C = 128
NCHUNK = BPW // C
G = C // 16


def _fuse_rel_tables(rel_cen, rel_off_raw):
    def body(cen_ref, off_ref, out_ref):
        z = off_ref[...]
        sp = jnp.log1p(jnp.exp(-jnp.abs(z))) + jnp.maximum(z, 0.0)
        out_ref[:, :D] = cen_ref[...]
        out_ref[:, D:] = (1.0 - ALPHA) * sp

    return pl.pallas_call(
        body,
        out_shape=jax.ShapeDtypeStruct((N_REL, 2 * D), jnp.float32),
    )(rel_cen, rel_off_raw)


def _softplus_vec(z):
    u = jnp.exp(-jnp.abs(z))
    s = u / (2.0 + u)
    s2 = s * s
    l1p = 2.0 * s * (1.0 + s2 * (1.0 / 3.0 + s2 * (1.0 / 5.0 + s2 * (1.0 / 7.0))))
    return l1p + jnp.maximum(z, 0.0)


def _make_sc_kernel():
    mesh = plsc.VectorSubcoreMesh(core_axis_name="c", subcore_axis_name="s")

    @functools.partial(
        pl.kernel,
        mesh=mesh,
        out_type=jax.ShapeDtypeStruct((B,), jnp.float32),
        scratch_types=[
            pltpu.VMEM((NCHUNK, C), jnp.int32),
            pltpu.VMEM((NCHUNK, C), jnp.int32),
            pltpu.VMEM((NCHUNK, C), jnp.int32),
            pltpu.VMEM((NCHUNK, C), jnp.int32),
            pltpu.VMEM((C, D), jnp.float32),
            pltpu.VMEM((C, D), jnp.float32),
            pltpu.VMEM((C, D), jnp.float32),
            pltpu.VMEM((C, 2 * D), jnp.float32),
            pltpu.VMEM((BPW,), jnp.float32),
            pltpu.SemaphoreType.DMA,
        ],
        compiler_params=pltpu.CompilerParams(
            needs_layout_passes=False, use_tc_tiling_on_sc=False
        ),
    )
    def k(head2d, pos2d, neg2d, rel2d, entity, reltab, out,
          hidx, pidx, nidx, ridx, hbuf, pbuf, nbuf, rbuf, obuf, sem):
        wid = lax.axis_index("s") * NC + lax.axis_index("c")
        row0 = wid * NCHUNK

        pltpu.sync_copy(head2d.at[pl.ds(row0, NCHUNK)], hidx)
        pltpu.sync_copy(pos2d.at[pl.ds(row0, NCHUNK)], pidx)
        pltpu.sync_copy(neg2d.at[pl.ds(row0, NCHUNK)], nidx)
        pltpu.sync_copy(rel2d.at[pl.ds(row0, NCHUNK)], ridx)

        iota = lax.iota(jnp.int32, 16)

        def chunk_body(j, _):
            cp_h = pltpu.make_async_copy(entity.at[hidx.at[j]], hbuf, sem)
            cp_p = pltpu.make_async_copy(entity.at[pidx.at[j]], pbuf, sem)
            cp_n = pltpu.make_async_copy(entity.at[nidx.at[j]], nbuf, sem)
            cp_r = pltpu.make_async_copy(reltab.at[ridx.at[j]], rbuf, sem)
            cp_h.start()
            cp_p.start()
            cp_n.start()
            cp_r.start()
            cp_h.wait()
            cp_p.wait()
            cp_n.wait()
            cp_r.wait()

            def group_body(g, _):
                rows = g * 16 + iota
                acc_p = jnp.zeros((16,), jnp.float32)
                acc_n = jnp.zeros((16,), jnp.float32)
                for d in range(D):
                    col = jnp.full((16,), d, jnp.int32)
                    h = plsc.load_gather(hbuf, [rows, col])
                    rc = plsc.load_gather(rbuf, [rows, col])
                    o = plsc.load_gather(rbuf, [rows, col + D])
                    p = plsc.load_gather(pbuf, [rows, col])
                    n = plsc.load_gather(nbuf, [rows, col])
                    c = h + rc
                    dp = jnp.abs(p - c)
                    dn = jnp.abs(n - c)
                    acc_p = acc_p + jnp.maximum(ALPHA * dp, dp - o)
                    acc_n = acc_n + jnp.maximum(ALPHA * dn, dn - o)
                score_p = GAMMA - acc_p
                score_n = GAMMA - acc_n
                loss = _softplus_vec(-score_p) + _softplus_vec(score_n)
                obuf[pl.ds(j * C + g * 16, 16)] = loss
                return 0

            lax.fori_loop(0, G, group_body, 0)
            return 0

        lax.fori_loop(0, NCHUNK, chunk_body, 0)
        pltpu.sync_copy(obuf, out.at[pl.ds(wid * BPW, BPW)])

    return k


_sc_kernel = _make_sc_kernel()


def kernel(head_idx, relation_idx, pos_idx, neg_idx, entity_emb, rel_cen, rel_off_raw):
    reltab = _fuse_rel_tables(rel_cen, rel_off_raw)
    head2d = head_idx.astype(jnp.int32).reshape(NW * NCHUNK, C)
    pos2d = pos_idx.astype(jnp.int32).reshape(NW * NCHUNK, C)
    neg2d = neg_idx.astype(jnp.int32).reshape(NW * NCHUNK, C)
    rel2d = relation_idx.astype(jnp.int32).reshape(NW * NCHUNK, C)
    return _sc_kernel(head2d, pos2d, neg2d, rel2d, entity_emb, reltab)


# speedup vs baseline: 1.8592x; 1.8592x over previous
import functools

import jax
import jax.numpy as jnp
from jax import lax
from jax.experimental import pallas as pl
from jax.experimental.pallas import tpu as pltpu
from jax.experimental.pallas import tpu_sc as plsc

M = 1000000
N_REL = 1000
D = 64
B = 16384
ALPHA = 0.2
GAMMA = 12.0

NC = 2
NS = 16
NW = NC * NS
BPW = B // NW
C = 128
NCHUNK = BPW // C
G = C // 16


def _fuse_rel_tables(rel_cen, rel_off_raw):
    def body(cen_ref, off_ref, out_ref):
        z = off_ref[...]
        sp = jnp.log1p(jnp.exp(-jnp.abs(z))) + jnp.maximum(z, 0.0)
        out_ref[:, :D] = cen_ref[...]
        out_ref[:, D:] = (1.0 - ALPHA) * sp

    return pl.pallas_call(
        body,
        out_shape=jax.ShapeDtypeStruct((N_REL, 2 * D), jnp.float32),
    )(rel_cen, rel_off_raw)


LB = 8192
_TGRID = -(-M // LB)
_T_ROWS = _TGRID * (LB // 2)


def _transpose_table(entity_t):
    def body(in_ref, out_ref):
        t = jnp.transpose(in_ref[...])
        out_ref[:, :D] = t[: LB // 2]
        out_ref[:, D:] = t[LB // 2 :]

    return pl.pallas_call(
        body,
        grid=(_TGRID,),
        in_specs=[pl.BlockSpec((D, LB), lambda i: (0, i))],
        out_specs=pl.BlockSpec((LB // 2, 2 * D), lambda i: (i, 0)),
        out_shape=jax.ShapeDtypeStruct((_T_ROWS, 2 * D), jnp.float32),
    )(entity_t)


def _softplus_vec(z):
    u = jnp.exp(-jnp.abs(z))
    s = u / (2.0 + u)
    s2 = s * s
    l1p = 2.0 * s * (1.0 + s2 * (1.0 / 3.0 + s2 * (1.0 / 5.0 + s2 * (1.0 / 7.0))))
    return l1p + jnp.maximum(z, 0.0)


def _make_sc_kernel():
    mesh = plsc.VectorSubcoreMesh(core_axis_name="c", subcore_axis_name="s")

    @functools.partial(
        pl.kernel,
        mesh=mesh,
        out_type=jax.ShapeDtypeStruct((B,), jnp.float32),
        scratch_types=[
            pltpu.VMEM((NCHUNK, C), jnp.int32),
            pltpu.VMEM((NCHUNK, C), jnp.int32),
            pltpu.VMEM((NCHUNK, C), jnp.int32),
            pltpu.VMEM((NCHUNK, C), jnp.int32),
            pltpu.VMEM((NCHUNK, C), jnp.int32),
            pltpu.VMEM((NCHUNK, C), jnp.int32),
            pltpu.VMEM((NCHUNK, C), jnp.int32),
            pltpu.VMEM((C, 2 * D), jnp.float32),
            pltpu.VMEM((C, 2 * D), jnp.float32),
            pltpu.VMEM((C, 2 * D), jnp.float32),
            pltpu.VMEM((C, 2 * D), jnp.float32),
            pltpu.VMEM((BPW,), jnp.float32),
            pltpu.SemaphoreType.DMA,
        ],
        compiler_params=pltpu.CompilerParams(needs_layout_passes=False),
    )
    def k(hpair2d, ppair2d, npair2d, rel2d, hoff2d, poff2d, noff2d,
          entity2, reltab, out,
          hidx, pidx, nidx, ridx, hoff, poff, noff,
          hbuf, pbuf, nbuf, rbuf, obuf, sem):
        wid = lax.axis_index("s") * NC + lax.axis_index("c")
        row0 = wid * NCHUNK

        pltpu.sync_copy(hpair2d.at[pl.ds(row0, NCHUNK)], hidx)
        pltpu.sync_copy(ppair2d.at[pl.ds(row0, NCHUNK)], pidx)
        pltpu.sync_copy(npair2d.at[pl.ds(row0, NCHUNK)], nidx)
        pltpu.sync_copy(rel2d.at[pl.ds(row0, NCHUNK)], ridx)
        pltpu.sync_copy(hoff2d.at[pl.ds(row0, NCHUNK)], hoff)
        pltpu.sync_copy(poff2d.at[pl.ds(row0, NCHUNK)], poff)
        pltpu.sync_copy(noff2d.at[pl.ds(row0, NCHUNK)], noff)

        iota = lax.iota(jnp.int32, 16)

        def chunk_body(j, _):
            cp_h = pltpu.make_async_copy(entity2.at[hidx.at[j]], hbuf, sem)
            cp_p = pltpu.make_async_copy(entity2.at[pidx.at[j]], pbuf, sem)
            cp_n = pltpu.make_async_copy(entity2.at[nidx.at[j]], nbuf, sem)
            cp_r = pltpu.make_async_copy(reltab.at[ridx.at[j]], rbuf, sem)
            cp_h.start()
            cp_p.start()
            cp_n.start()
            cp_r.start()
            cp_h.wait()
            cp_p.wait()
            cp_n.wait()
            cp_r.wait()

            def group_body(g, _):
                rows = g * 16 + iota
                ch = hoff[j, pl.ds(g * 16, 16)]
                cpo = poff[j, pl.ds(g * 16, 16)]
                cn = noff[j, pl.ds(g * 16, 16)]
                acc_p = jnp.zeros((16,), jnp.float32)
                acc_n = jnp.zeros((16,), jnp.float32)
                for d in range(D):
                    col = jnp.full((16,), d, jnp.int32)
                    h = plsc.load_gather(hbuf, [rows, ch + d])
                    rc = plsc.load_gather(rbuf, [rows, col])
                    o = plsc.load_gather(rbuf, [rows, col + D])
                    p = plsc.load_gather(pbuf, [rows, cpo + d])
                    n = plsc.load_gather(nbuf, [rows, cn + d])
                    c = h + rc
                    dp = jnp.abs(p - c)
                    dn = jnp.abs(n - c)
                    acc_p = acc_p + jnp.maximum(ALPHA * dp, dp - o)
                    acc_n = acc_n + jnp.maximum(ALPHA * dn, dn - o)
                score_p = GAMMA - acc_p
                score_n = GAMMA - acc_n
                loss = _softplus_vec(-score_p) + _softplus_vec(score_n)
                obuf[pl.ds(j * C + g * 16, 16)] = loss
                return 0

            lax.fori_loop(0, G, group_body, 0)
            return 0

        lax.fori_loop(0, NCHUNK, chunk_body, 0)
        pltpu.sync_copy(obuf, out.at[pl.ds(wid * BPW, BPW)])

    return k


_sc_kernel = _make_sc_kernel()


def kernel(head_idx, relation_idx, pos_idx, neg_idx, entity_emb, rel_cen, rel_off_raw):
    reltab = _fuse_rel_tables(rel_cen, rel_off_raw)
    entity2 = _transpose_table(entity_emb.T)

    def prep(ix):
        ix = ix.astype(jnp.int32)
        pair = (((ix >> 13) << 12) + (ix & 4095)).reshape(NW * NCHUNK, C)
        off = (((ix >> 12) & 1) * D).reshape(NW * NCHUNK, C)
        return pair, off

    hpair, hoff = prep(head_idx)
    ppair, poff = prep(pos_idx)
    npair, noff = prep(neg_idx)
    rel2d = relation_idx.astype(jnp.int32).reshape(NW * NCHUNK, C)
    return _sc_kernel(hpair, ppair, npair, rel2d, hoff, poff, noff,
                      entity2, reltab)


# speedup vs baseline: 2.6298x; 1.4144x over previous
import functools

import jax
import jax.numpy as jnp
from jax import lax
from jax.experimental import pallas as pl
from jax.experimental.pallas import tpu as pltpu
from jax.experimental.pallas import tpu_sc as plsc

M = 1000000
N_REL = 1000
D = 64
B = 16384
ALPHA = 0.2
GAMMA = 12.0

NC = 2
NS = 16
NW = NC * NS
BPW = B // NW
C = 64
NCHUNK = BPW // C
G = C // 16


def _fuse_rel_tables(rel_cen, rel_off_raw):
    def body(cen_ref, off_ref, out_ref):
        z = off_ref[...]
        sp = jnp.log1p(jnp.exp(-jnp.abs(z))) + jnp.maximum(z, 0.0)
        out_ref[:, :D] = cen_ref[...]
        out_ref[:, D:] = (1.0 - ALPHA) * sp

    return pl.pallas_call(
        body,
        out_shape=jax.ShapeDtypeStruct((N_REL, 2 * D), jnp.float32),
    )(rel_cen, rel_off_raw)


LB = 32768
_LBH_SHIFT = 14
_TGRID = -(-M // LB)
_T_ROWS = _TGRID * (LB // 2)


def _transpose_table(entity_t):
    def body(in_ref, out_ref):
        t = jnp.transpose(in_ref[...])
        out_ref[:, :D] = t[: LB // 2]
        out_ref[:, D:] = t[LB // 2 :]

    return pl.pallas_call(
        body,
        grid=(_TGRID,),
        in_specs=[pl.BlockSpec((D, LB), lambda i: (0, i))],
        out_specs=pl.BlockSpec((LB // 2, 2 * D), lambda i: (i, 0)),
        out_shape=jax.ShapeDtypeStruct((_T_ROWS, 2 * D), jnp.float32),
    )(entity_t)


def _softplus_vec(z):
    u = jnp.exp(-jnp.abs(z))
    s = u / (2.0 + u)
    s2 = s * s
    l1p = 2.0 * s * (1.0 + s2 * (1.0 / 3.0 + s2 * (1.0 / 5.0 + s2 * (1.0 / 7.0))))
    return l1p + jnp.maximum(z, 0.0)


def _make_sc_kernel():
    mesh = plsc.VectorSubcoreMesh(core_axis_name="c", subcore_axis_name="s")

    @functools.partial(
        pl.kernel,
        mesh=mesh,
        out_type=jax.ShapeDtypeStruct((B,), jnp.float32),
        scratch_types=[
            pltpu.VMEM((NCHUNK, C), jnp.int32),
            pltpu.VMEM((NCHUNK, C), jnp.int32),
            pltpu.VMEM((NCHUNK, C), jnp.int32),
            pltpu.VMEM((NCHUNK, C), jnp.int32),
            pltpu.VMEM((NCHUNK, C), jnp.int32),
            pltpu.VMEM((NCHUNK, C), jnp.int32),
            pltpu.VMEM((NCHUNK, C), jnp.int32),
            pltpu.VMEM((C, 2 * D), jnp.float32),
            pltpu.VMEM((C, 2 * D), jnp.float32),
            pltpu.VMEM((C, 2 * D), jnp.float32),
            pltpu.VMEM((C, 2 * D), jnp.float32),
            pltpu.VMEM((C, 2 * D), jnp.float32),
            pltpu.VMEM((C, 2 * D), jnp.float32),
            pltpu.VMEM((C, 2 * D), jnp.float32),
            pltpu.VMEM((C, 2 * D), jnp.float32),
            pltpu.VMEM((BPW,), jnp.float32),
            pltpu.SemaphoreType.DMA,
            pltpu.SemaphoreType.DMA,
        ],
        compiler_params=pltpu.CompilerParams(needs_layout_passes=False),
    )
    def k(hpair2d, ppair2d, npair2d, rel2d, hoff2d, poff2d, noff2d,
          entity2, reltab, out,
          hidx, pidx, nidx, ridx, hoff, poff, noff,
          hbuf0, pbuf0, nbuf0, rbuf0, hbuf1, pbuf1, nbuf1, rbuf1,
          obuf, sem0, sem1):
        wid = lax.axis_index("s") * NC + lax.axis_index("c")
        row0 = wid * NCHUNK

        pltpu.sync_copy(hpair2d.at[pl.ds(row0, NCHUNK)], hidx)
        pltpu.sync_copy(ppair2d.at[pl.ds(row0, NCHUNK)], pidx)
        pltpu.sync_copy(npair2d.at[pl.ds(row0, NCHUNK)], nidx)
        pltpu.sync_copy(rel2d.at[pl.ds(row0, NCHUNK)], ridx)
        pltpu.sync_copy(hoff2d.at[pl.ds(row0, NCHUNK)], hoff)
        pltpu.sync_copy(poff2d.at[pl.ds(row0, NCHUNK)], poff)
        pltpu.sync_copy(noff2d.at[pl.ds(row0, NCHUNK)], noff)

        iota = lax.iota(jnp.int32, 16)
        bufs = ((hbuf0, pbuf0, nbuf0, rbuf0, sem0),
                (hbuf1, pbuf1, nbuf1, rbuf1, sem1))

        def start_chunk(j, b):
            hb, pb, nb, rb, sem = bufs[b]
            pltpu.make_async_copy(entity2.at[hidx.at[j]], hb, sem).start()
            pltpu.make_async_copy(entity2.at[pidx.at[j]], pb, sem).start()
            pltpu.make_async_copy(entity2.at[nidx.at[j]], nb, sem).start()
            pltpu.make_async_copy(reltab.at[ridx.at[j]], rb, sem).start()

        def wait_chunk(b):
            hb, pb, nb, rb, sem = bufs[b]
            pltpu.make_async_copy(entity2.at[hidx.at[0]], hb, sem).wait()
            pltpu.make_async_copy(entity2.at[pidx.at[0]], pb, sem).wait()
            pltpu.make_async_copy(entity2.at[nidx.at[0]], nb, sem).wait()
            pltpu.make_async_copy(reltab.at[ridx.at[0]], rb, sem).wait()

        def compute_chunk(j, b):
            hb, pb, nb, rb, _ = bufs[b]

            def group_body(g, _):
                rows = g * 16 + iota
                chv = hoff[j, pl.ds(g * 16, 16)]
                cpv = poff[j, pl.ds(g * 16, 16)]
                cnv = noff[j, pl.ds(g * 16, 16)]
                acc_p = jnp.zeros((16,), jnp.float32)
                acc_n = jnp.zeros((16,), jnp.float32)
                for d in range(D):
                    rot = (iota + d) & 63
                    h = plsc.load_gather(hb, [rows, chv + rot])
                    rc = plsc.load_gather(rb, [rows, rot])
                    o = plsc.load_gather(rb, [rows, rot + D])
                    p = plsc.load_gather(pb, [rows, cpv + rot])
                    n = plsc.load_gather(nb, [rows, cnv + rot])
                    c = h + rc
                    dp = jnp.abs(p - c)
                    dn = jnp.abs(n - c)
                    acc_p = acc_p + jnp.maximum(ALPHA * dp, dp - o)
                    acc_n = acc_n + jnp.maximum(ALPHA * dn, dn - o)
                score_p = GAMMA - acc_p
                score_n = GAMMA - acc_n
                loss = _softplus_vec(-score_p) + _softplus_vec(score_n)
                obuf[pl.ds(j * C + g * 16, 16)] = loss
                return 0

            lax.fori_loop(0, G, group_body, 0)

        start_chunk(0, 0)
        start_chunk(1, 1)

        def pair_body(kk, _):
            for b in range(2):
                j = 2 * kk + b
                wait_chunk(b)
                compute_chunk(j, b)

                @pl.when(j + 2 < NCHUNK)
                def _():
                    start_chunk(j + 2, b)

            return 0

        lax.fori_loop(0, NCHUNK // 2, pair_body, 0)
        pltpu.sync_copy(obuf, out.at[pl.ds(wid * BPW, BPW)])

    return k


_sc_kernel = _make_sc_kernel()


def kernel(head_idx, relation_idx, pos_idx, neg_idx, entity_emb, rel_cen, rel_off_raw):
    reltab = _fuse_rel_tables(rel_cen, rel_off_raw)
    entity2 = _transpose_table(entity_emb.T)

    def prep(ix):
        ix = ix.astype(jnp.int32)
        half = LB // 2
        pair = (((ix >> (_LBH_SHIFT + 1)) << _LBH_SHIFT)
                + (ix & (half - 1))).reshape(NW * NCHUNK, C)
        off = (((ix >> _LBH_SHIFT) & 1) * D).reshape(NW * NCHUNK, C)
        return pair, off

    hpair, hoff = prep(head_idx)
    ppair, poff = prep(pos_idx)
    npair, noff = prep(neg_idx)
    rel2d = relation_idx.astype(jnp.int32).reshape(NW * NCHUNK, C)
    return _sc_kernel(hpair, ppair, npair, rel2d, hoff, poff, noff,
                      entity2, reltab)
